# baseline (speedup 1.0000x reference)
import functools
import jax
import jax.numpy as jnp
from jax.experimental import pallas as pl
from jax.experimental.pallas import tpu as pltpu

N = 4096
B = 8
K = 5
G = 10
C = 10
RB = 1024


def _apply_first_body(L_ref, t_ref, out_ref):
    out_ref[...] = jnp.dot(L_ref[...], t_ref[...],
                           preferred_element_type=jnp.float32)


def _apply_rec_body(L_ref, t_ref, tprev_ref, out_ref):
    out_ref[...] = 2.0 * jnp.dot(L_ref[...], t_ref[...],
                                 preferred_element_type=jnp.float32) \
        - tprev_ref[...]


def _lapply(L, t):
    cols = t.shape[1]
    return pl.pallas_call(
        _apply_first_body,
        grid=(N // RB,),
        in_specs=[
            pl.BlockSpec((RB, N), lambda i: (i, 0)),
            pl.BlockSpec((N, cols), lambda i: (0, 0)),
        ],
        out_specs=pl.BlockSpec((RB, cols), lambda i: (i, 0)),
        out_shape=jax.ShapeDtypeStruct((N, cols), jnp.float32),
    )(L, t)


def _lapply_rec(L, t, tprev):
    cols = t.shape[1]
    return pl.pallas_call(
        _apply_rec_body,
        grid=(N // RB,),
        in_specs=[
            pl.BlockSpec((RB, N), lambda i: (i, 0)),
            pl.BlockSpec((N, cols), lambda i: (0, 0)),
            pl.BlockSpec((RB, cols), lambda i: (i, 0)),
        ],
        out_specs=pl.BlockSpec((RB, cols), lambda i: (i, 0)),
        out_shape=jax.ShapeDtypeStruct((N, cols), jnp.float32),
    )(L, t, tprev)


def _mix_body(t0, t1, t2, t3, t4, w_ref, b_ref, out_ref):
    acc = jnp.dot(t0[...], w_ref[0], preferred_element_type=jnp.float32)
    acc += jnp.dot(t1[...], w_ref[1], preferred_element_type=jnp.float32)
    acc += jnp.dot(t2[...], w_ref[2], preferred_element_type=jnp.float32)
    acc += jnp.dot(t3[...], w_ref[3], preferred_element_type=jnp.float32)
    acc += jnp.dot(t4[...], w_ref[4], preferred_element_type=jnp.float32)
    out_ref[...] = jax.nn.relu(acc + b_ref[...])


def _mix(ts, w_bd, bias):
    cin = ts[0].shape[1]
    cout = w_bd.shape[2]
    return pl.pallas_call(
        _mix_body,
        out_shape=jax.ShapeDtypeStruct((N, cout), jnp.float32),
    )(*ts, w_bd, bias)


def _fc_body(h_ref, fcw_ref, fcb_ref, out_ref):
    logits = jnp.dot(h_ref[...], fcw_ref[...],
                     preferred_element_type=jnp.float32) + fcb_ref[...]
    m = jnp.max(logits, axis=1, keepdims=True)
    s = jnp.log(jnp.sum(jnp.exp(logits - m), axis=1, keepdims=True))
    out_ref[...] = logits - (m + s)


def _cheb_layer(L, t0, w_bd, bias):
    ts = [t0, _lapply(L, t0)]
    for _ in range(2, K):
        ts.append(_lapply_rec(L, ts[-1], ts[-2]))
    return _mix(ts, w_bd, bias)


@jax.jit
def kernel(x, L, W1, b1, W2, b2, W3, b3, fc_w, fc_b):
    x0 = x[:, :, 0].T
    eyeB = jnp.eye(B, dtype=jnp.float32)
    w1_bd = jnp.einsum('ab,kfg->kafbg', eyeB, W1).reshape(K, B, B * G)
    w2_bd = jnp.einsum('ab,kfg->kafbg', eyeB, W2).reshape(K, B * G, B * G)
    w3_bd = jnp.einsum('ab,kfg->kafbg', eyeB, W3).reshape(K, B * G, B * G)
    bb1 = jnp.tile(b1, B)[None, :]
    bb2 = jnp.tile(b2, B)[None, :]
    bb3 = jnp.tile(b3, B)[None, :]

    h = _cheb_layer(L, x0, w1_bd, bb1)
    h = _cheb_layer(L, h, w2_bd, bb2)
    h = _cheb_layer(L, h, w3_bd, bb3)

    ht = h.reshape(N, B, G).transpose(1, 0, 2).reshape(B, N * G)
    return pl.pallas_call(
        _fc_body,
        out_shape=jax.ShapeDtypeStruct((B, C), jnp.float32),
    )(ht, fc_w, fc_b[None, :])


# speedup vs baseline: 1.2573x; 1.2573x over previous
import functools
import jax
import jax.numpy as jnp
from jax.experimental import pallas as pl
from jax.experimental.pallas import tpu as pltpu

N = 4096
B = 8
K = 5
G = 10
C = 10
RB = 1024


def _apply_first_body(L_ref, t_ref, out_ref):
    out_ref[...] = jnp.dot(L_ref[...], t_ref[...].astype(jnp.bfloat16),
                           preferred_element_type=jnp.float32)


def _apply_rec_body(L_ref, t_ref, tprev_ref, out_ref):
    out_ref[...] = 2.0 * jnp.dot(L_ref[...],
                                 t_ref[...].astype(jnp.bfloat16),
                                 preferred_element_type=jnp.float32) \
        - tprev_ref[...]


def _lapply(L, t):
    cols = t.shape[1]
    return pl.pallas_call(
        _apply_first_body,
        grid=(N // RB,),
        in_specs=[
            pl.BlockSpec((RB, N), lambda i: (i, 0)),
            pl.BlockSpec((N, cols), lambda i: (0, 0)),
        ],
        out_specs=pl.BlockSpec((RB, cols), lambda i: (i, 0)),
        out_shape=jax.ShapeDtypeStruct((N, cols), jnp.float32),
    )(L, t)


def _lapply_rec(L, t, tprev):
    cols = t.shape[1]
    return pl.pallas_call(
        _apply_rec_body,
        grid=(N // RB,),
        in_specs=[
            pl.BlockSpec((RB, N), lambda i: (i, 0)),
            pl.BlockSpec((N, cols), lambda i: (0, 0)),
            pl.BlockSpec((RB, cols), lambda i: (i, 0)),
        ],
        out_specs=pl.BlockSpec((RB, cols), lambda i: (i, 0)),
        out_shape=jax.ShapeDtypeStruct((N, cols), jnp.float32),
    )(L, t, tprev)


def _mix_body(t0, t1, t2, t3, t4, w_ref, b_ref, out_ref):
    acc = jnp.dot(t0[...], w_ref[0], preferred_element_type=jnp.float32)
    acc += jnp.dot(t1[...], w_ref[1], preferred_element_type=jnp.float32)
    acc += jnp.dot(t2[...], w_ref[2], preferred_element_type=jnp.float32)
    acc += jnp.dot(t3[...], w_ref[3], preferred_element_type=jnp.float32)
    acc += jnp.dot(t4[...], w_ref[4], preferred_element_type=jnp.float32)
    out_ref[...] = jax.nn.relu(acc + b_ref[...])


def _mix(ts, w_bd, bias):
    cin = ts[0].shape[1]
    cout = w_bd.shape[2]
    return pl.pallas_call(
        _mix_body,
        out_shape=jax.ShapeDtypeStruct((N, cout), jnp.float32),
    )(*ts, w_bd, bias)


def _fc_body(h_ref, fcw_ref, fcb_ref, out_ref):
    logits = jnp.dot(h_ref[...], fcw_ref[...],
                     preferred_element_type=jnp.float32) + fcb_ref[...]
    m = jnp.max(logits, axis=1, keepdims=True)
    s = jnp.log(jnp.sum(jnp.exp(logits - m), axis=1, keepdims=True))
    out_ref[...] = logits - (m + s)


def _cheb_layer(L, t0, w_bd, bias):
    ts = [t0, _lapply(L, t0)]
    for _ in range(2, K):
        ts.append(_lapply_rec(L, ts[-1], ts[-2]))
    return _mix(ts, w_bd, bias)


@jax.jit
def kernel(x, L, W1, b1, W2, b2, W3, b3, fc_w, fc_b):
    x0 = x[:, :, 0].T
    Lb = L.astype(jnp.bfloat16)
    eyeB = jnp.eye(B, dtype=jnp.float32)
    w1_bd = jnp.einsum('ab,kfg->kafbg', eyeB, W1).reshape(K, B, B * G)
    w2_bd = jnp.einsum('ab,kfg->kafbg', eyeB, W2).reshape(K, B * G, B * G)
    w3_bd = jnp.einsum('ab,kfg->kafbg', eyeB, W3).reshape(K, B * G, B * G)
    bb1 = jnp.tile(b1, B)[None, :]
    bb2 = jnp.tile(b2, B)[None, :]
    bb3 = jnp.tile(b3, B)[None, :]

    h = _cheb_layer(Lb, x0, w1_bd, bb1)
    h = _cheb_layer(Lb, h, w2_bd, bb2)
    h = _cheb_layer(Lb, h, w3_bd, bb3)

    ht = h.reshape(N, B, G).transpose(1, 0, 2).reshape(B, N * G)
    return pl.pallas_call(
        _fc_body,
        out_shape=jax.ShapeDtypeStruct((B, C), jnp.float32),
    )(ht, fc_w, fc_b[None, :])
